# baseline (speedup 1.0000x reference)
import jax
import jax.numpy as jnp
from jax import lax
from jax.experimental import pallas as pl
from jax.experimental.pallas import tpu as pltpu
from jax.experimental.pallas import tpu_sc as plsc

NC = 2
NS = 16
L = 16
NW = NC * NS
N = 128 * 8192
SHARD = N // NW
ITERS = SHARD // L
KK = 8192
NB12 = 4096
NB3 = 256

_MESH = plsc.VectorSubcoreMesh(
    core_axis_name="c", subcore_axis_name="s", num_cores=NC, num_subcores=NS)
_SC_PARAMS = pltpu.CompilerParams(needs_layout_passes=False)


def _bits_vec(xv):
    return jnp.maximum(plsc.bitcast(xv, jnp.int32), 0)


def _body(x_hbm, out_hbm, xchg_hbm, h3x_hbm, xs, hist, tmp, h3all, keys, idxs,
          shared, shared_red, gpref, sem):
    cid = lax.axis_index("c")
    sid = lax.axis_index("s")
    wid = sid * NC + cid
    pltpu.sync_copy(x_hbm.at[pl.ds(wid * SHARD, SHARD)], xs)
    ones = jnp.ones((L,), jnp.int32)

    def zero_hist(nbins):
        z = jnp.zeros((L,), jnp.int32)

        def zb(i, c):
            hist[pl.ds(i * L, L)] = z
            return c

        lax.fori_loop(0, nbins // L, zb, 0, unroll=4)

    def hist1_pass():
        zero_hist(NB12)

        zv = jnp.zeros((L,), jnp.int32)

        def hb(i, c):
            xv = xs[pl.ds(i * L, L)]
            bits = _bits_vec(xv)
            plsc.addupdate_scatter(hist, [bits >> 20], ones, mask=bits > zv)
            return c

        lax.fori_loop(0, ITERS, hb, 0, unroll=8)

    def compact_hist_pass(nbins, shift, mask_shift, selp, mi):
        zero_hist(nbins)
        selv = jnp.full((L,), selp, jnp.int32)

        def hb(i, c):
            kv = keys[pl.ds(i * L, L)]
            mask = (kv >> mask_shift) == selv
            plsc.addupdate_scatter(hist, [(kv >> shift) & (nbins - 1)], ones,
                                   mask=mask)
            return c

        lax.fori_loop(0, mi, hb, 0)

    def reduce_exchange(nbins):
        nred = nbins // NS
        pltpu.sync_copy(hist.at[pl.ds(0, nbins)],
                        shared.at[pl.ds(sid * NB12, nbins)])
        plsc.subcore_barrier()
        for r in range(NS):
            pltpu.sync_copy(shared.at[pl.ds(r * NB12 + sid * nred, nred)],
                            tmp.at[pl.ds(r * nred, nred)])
        for g in range(nred // L):
            def rb(r, acc):
                return acc + tmp[pl.ds(r * nred + g * L, L)]

            hist[pl.ds(g * L, L)] = lax.fori_loop(
                0, NS, rb, jnp.zeros((L,), jnp.int32), unroll=4)
        pltpu.sync_copy(hist.at[pl.ds(0, nred)],
                        shared_red.at[pl.ds(sid * nred, nred)])
        plsc.subcore_barrier()

        @pl.when(sid == 0)
        def _():
            pltpu.sync_copy(shared_red.at[pl.ds(0, nbins)],
                            xchg_hbm.at[pl.ds(cid * NB12, nbins)])

        pltpu.core_barrier(sem, core_axis_name="c")
        plsc.subcore_barrier()
        pltpu.sync_copy(shared_red.at[pl.ds(0, nbins)], tmp.at[pl.ds(0, nbins)])
        pltpu.sync_copy(xchg_hbm.at[pl.ds((1 - cid) * NB12, nbins)],
                        hist.at[pl.ds(0, nbins)])

        def cb(i, c):
            v = tmp[pl.ds(i * L, L)] + hist[pl.ds(i * L, L)]
            tmp[pl.ds(i * L, L)] = plsc.cumsum(v)
            return c

        lax.fori_loop(0, nbins // L, cb, 0, unroll=4)

    def group_prefix(nbins):
        ng = nbins // L

        def gb(g, acc):
            gpref[g] = acc
            return acc + tmp[pl.ds(g * L, L)][L - 1]

        return lax.fori_loop(0, ng, gb, jnp.int32(0))

    def make_count_ge(nbins, tot):
        ng = nbins // L

        def count_ge(b):
            g = jnp.minimum(b >> 4, ng - 1)
            r = b & (L - 1)
            tprev = tmp[pl.ds(jnp.maximum(b - 1, 0), L)][0]
            pe = gpref[g] + jnp.where(r > 0, tprev, 0)
            return jnp.where(b >= nbins, 0, tot - pe)

        return count_ge

    def bsearch(count_ge, nbits, r):
        def sb(k, b):
            cand = b + lax.shift_left(jnp.int32(1), nbits - 1 - k)
            return jnp.where(count_ge(cand) >= r, cand, b)

        b = lax.fori_loop(0, nbits, sb, jnp.int32(0))
        return b, count_ge(b + 1)

    hist1_pass()
    reduce_exchange(NB12)
    tot1 = group_prefix(NB12)
    cg1 = make_count_ge(NB12, tot1)
    b1, above1 = bsearch(cg1, 12, jnp.int32(KK))

    b1v = jnp.full((L,), b1, jnp.int32)
    iota0 = lax.broadcasted_iota(jnp.int32, (L,), 0)

    zf = jnp.zeros((L,), jnp.float32)

    def cp(i, cnt):
        xv = xs[pl.ds(i * L, L)]
        key = _bits_vec(xv)
        mge = (key >> 20) >= b1v
        mgi = jnp.where(mge, 1, 0)
        off = cnt + plsc.cumsum(mgi) - mgi
        plsc.store_scatter(keys, [off], key, mask=mge)
        plsc.store_scatter(idxs, [off], iota0 + i * L, mask=mge)
        xs[pl.ds(i * L, L)] = zf
        return cnt + plsc.all_reduce_population_count(mge)

    cntv = lax.fori_loop(0, ITERS, cp, jnp.zeros((L,), jnp.int32), unroll=8)
    m = cntv[0]
    keys[pl.ds(m, L)] = jnp.full((L,), -1, jnp.int32)
    mi = (m + L - 1) // L

    compact_hist_pass(NB12, 8, 20, b1, mi)
    reduce_exchange(NB12)
    tot2 = group_prefix(NB12)
    cg2 = make_count_ge(NB12, tot2)
    b2, sfx2 = bsearch(cg2, 12, KK - above1)
    p24 = lax.shift_left(b1, 12) | b2
    above2 = above1 + sfx2

    compact_hist_pass(NB3, 0, 8, p24, mi)
    pltpu.sync_copy(hist.at[pl.ds(0, NB3)], h3x_hbm.at[pl.ds(wid * NB3, NB3)])
    pltpu.core_barrier(sem, core_axis_name="c")
    plsc.subcore_barrier()
    pltpu.sync_copy(h3x_hbm, h3all.at[pl.ds(0, NW * NB3)])
    for g in range(NB3 // L):
        def ab(w, acc):
            return acc + h3all[pl.ds(w * NB3 + g * L, L)]

        v = lax.fori_loop(0, NW, ab, jnp.zeros((L,), jnp.int32), unroll=4)
        tmp[pl.ds(g * L, L)] = plsc.cumsum(v)
    tot3 = group_prefix(NB3)
    cg3 = make_count_ge(NB3, tot3)
    b3, sfx3 = bsearch(cg3, 8, KK - above2)
    count_greater = above2 + sfx3
    t_bits = lax.shift_left(p24, 8) | b3
    need = KK - count_greater

    def pb(w, acc):
        cnt = h3all[pl.ds(w * NB3 + b3, L)][0]
        return acc + jnp.where(w < wid, cnt, 0)

    base = lax.fori_loop(0, NW, pb, jnp.int32(0))

    tv = jnp.full((L,), t_bits, jnp.int32)
    needv = jnp.full((L,), need, jnp.int32)
    basev = jnp.full((L,), base, jnp.int32)

    def ws(i, carry):
        kv = keys[pl.ds(i * L, L)]
        iv = idxs[pl.ds(i * L, L)]
        gt = kv > tv
        eq = kv == tv
        eqi = jnp.where(eq, 1, 0)
        excl = plsc.cumsum(eqi) - eqi
        rank = basev + carry + excl
        keep = gt | (eq & (rank < needv))
        plsc.store_scatter(xs, [iv], plsc.bitcast(kv, jnp.float32), mask=keep)
        return carry + plsc.all_reduce_population_count(eq)

    lax.fori_loop(0, mi, ws, jnp.zeros((L,), jnp.int32))

    pltpu.sync_copy(xs, out_hbm.at[pl.ds(wid * SHARD, SHARD)])


_fused = pl.kernel(
    _body,
    out_type=(jax.ShapeDtypeStruct((N,), jnp.float32),
              jax.ShapeDtypeStruct((NC * NB12,), jnp.int32),
              jax.ShapeDtypeStruct((NW * NB3,), jnp.int32)),
    mesh=_MESH,
    scratch_types=[pltpu.VMEM((SHARD,), jnp.float32),
                   pltpu.VMEM((NB12,), jnp.int32),
                   pltpu.VMEM((NB12 + L,), jnp.int32),
                   pltpu.VMEM((NW * NB3 + L,), jnp.int32),
                   pltpu.VMEM((SHARD + L,), jnp.int32),
                   pltpu.VMEM((SHARD + L,), jnp.int32),
                   pltpu.VMEM_SHARED((NS * NB12,), jnp.int32),
                   pltpu.VMEM_SHARED((NB12,), jnp.int32),
                   pltpu.SMEM((NB12 // L,), jnp.int32),
                   pltpu.SemaphoreType.REGULAR],
    compiler_params=_SC_PARAMS,
    name="sc_topk_fused",
)


def kernel(x):
    out, _, _ = _fused(x.reshape(-1))
    return out.reshape(x.shape)


# speedup vs baseline: 1.0894x; 1.0894x over previous
import jax
import jax.numpy as jnp
from jax import lax
from jax.experimental import pallas as pl
from jax.experimental.pallas import tpu as pltpu
from jax.experimental.pallas import tpu_sc as plsc

NC = 2
NS = 16
L = 16
NW = NC * NS
N = 128 * 8192
SHARD = N // NW
ITERS = SHARD // L
KK = 8192
NB12 = 4096
NB3 = 256

_MESH = plsc.VectorSubcoreMesh(
    core_axis_name="c", subcore_axis_name="s", num_cores=NC, num_subcores=NS)
_SC_PARAMS = pltpu.CompilerParams(needs_layout_passes=False)


def _bits_vec(xv):
    return jnp.maximum(plsc.bitcast(xv, jnp.int32), 0)


def _body(x_hbm, out_hbm, xchg_hbm, h3x_hbm, xs, hist, tmp, h3all, keys, idxs,
          shared, shared_red, gpref, sem):
    cid = lax.axis_index("c")
    sid = lax.axis_index("s")
    wid = sid * NC + cid
    pltpu.sync_copy(x_hbm.at[pl.ds(wid * SHARD, SHARD)], xs)
    ones = jnp.ones((L,), jnp.int32)

    def zero_hist(nbins):
        z = jnp.zeros((L,), jnp.int32)

        def zb(i, c):
            hist[pl.ds(i * L, L)] = z
            return c

        lax.fori_loop(0, nbins // L, zb, 0, unroll=4)

    def hist1_pass():
        zero_hist(NB12)

        zv = jnp.zeros((L,), jnp.int32)

        def hb(i, c):
            xv = xs[pl.ds(i * L, L)]
            bits = _bits_vec(xv)
            plsc.addupdate_scatter(hist, [bits >> 20], ones, mask=bits > zv)
            return c

        lax.fori_loop(0, ITERS, hb, 0, unroll=8)

    def compact_hist_pass(nbins, shift, mask_shift, selp, mi):
        zero_hist(nbins)
        selv = jnp.full((L,), selp, jnp.int32)

        def hb(i, c):
            kv = keys[pl.ds(i * L, L)]
            mask = (kv >> mask_shift) == selv
            plsc.addupdate_scatter(hist, [(kv >> shift) & (nbins - 1)], ones,
                                   mask=mask)
            return c

        lax.fori_loop(0, mi, hb, 0)

    def reduce_exchange(nbins):
        nred = nbins // NS
        pltpu.sync_copy(hist.at[pl.ds(0, nbins)],
                        shared.at[pl.ds(sid * NB12, nbins)])
        plsc.subcore_barrier()
        for r in range(NS):
            pltpu.sync_copy(shared.at[pl.ds(r * NB12 + sid * nred, nred)],
                            tmp.at[pl.ds(r * nred, nred)])
        for g in range(nred // L):
            def rb(r, acc):
                return acc + tmp[pl.ds(r * nred + g * L, L)]

            hist[pl.ds(g * L, L)] = lax.fori_loop(
                0, NS, rb, jnp.zeros((L,), jnp.int32), unroll=4)
        pltpu.sync_copy(hist.at[pl.ds(0, nred)],
                        shared_red.at[pl.ds(sid * nred, nred)])
        plsc.subcore_barrier()

        @pl.when(sid == 0)
        def _():
            pltpu.sync_copy(shared_red.at[pl.ds(0, nbins)],
                            xchg_hbm.at[pl.ds(cid * NB12, nbins)])

        pltpu.core_barrier(sem, core_axis_name="c")
        plsc.subcore_barrier()
        pltpu.sync_copy(shared_red.at[pl.ds(0, nbins)], tmp.at[pl.ds(0, nbins)])
        pltpu.sync_copy(xchg_hbm.at[pl.ds((1 - cid) * NB12, nbins)],
                        hist.at[pl.ds(0, nbins)])

        def cb(i, c):
            v = tmp[pl.ds(i * L, L)] + hist[pl.ds(i * L, L)]
            tmp[pl.ds(i * L, L)] = plsc.cumsum(v)
            return c

        lax.fori_loop(0, nbins // L, cb, 0, unroll=4)

    def group_prefix(nbins):
        ng = nbins // L

        def gb(g, acc):
            gpref[g] = acc
            return acc + tmp[pl.ds(g * L, L)][L - 1]

        return lax.fori_loop(0, ng, gb, jnp.int32(0))

    def make_count_ge(nbins, tot):
        ng = nbins // L

        def count_ge(b):
            g = jnp.minimum(b >> 4, ng - 1)
            r = b & (L - 1)
            tprev = tmp[pl.ds(jnp.maximum(b - 1, 0), L)][0]
            pe = gpref[g] + jnp.where(r > 0, tprev, 0)
            return jnp.where(b >= nbins, 0, tot - pe)

        return count_ge

    def bsearch(count_ge, nbits, r):
        def sb(k, b):
            cand = b + lax.shift_left(jnp.int32(1), nbits - 1 - k)
            return jnp.where(count_ge(cand) >= r, cand, b)

        b = lax.fori_loop(0, nbits, sb, jnp.int32(0))
        return b, count_ge(b + 1)

    hist1_pass()
    reduce_exchange(NB12)
    tot1 = group_prefix(NB12)
    cg1 = make_count_ge(NB12, tot1)
    b1, above1 = bsearch(cg1, 12, jnp.int32(KK))

    b1v = jnp.full((L,), b1, jnp.int32)
    iota0 = lax.broadcasted_iota(jnp.int32, (L,), 0)

    zf = jnp.zeros((L,), jnp.float32)

    def cp(i, cnt):
        xv = xs[pl.ds(i * L, L)]
        key = _bits_vec(xv)
        mge = (key >> 20) >= b1v
        c0 = cnt[0]
        plsc.store_compressed(keys.at[pl.ds(c0, L)], key, mask=mge)
        plsc.store_compressed(idxs.at[pl.ds(c0, L)], iota0 + i * L, mask=mge)
        xs[pl.ds(i * L, L)] = zf
        return cnt + plsc.all_reduce_population_count(mge)

    cntv = lax.fori_loop(0, ITERS, cp, jnp.zeros((L,), jnp.int32), unroll=8)
    m = cntv[0]
    keys[pl.ds(m, L)] = jnp.full((L,), -1, jnp.int32)
    mi = (m + L - 1) // L

    compact_hist_pass(NB12, 8, 20, b1, mi)
    reduce_exchange(NB12)
    tot2 = group_prefix(NB12)
    cg2 = make_count_ge(NB12, tot2)
    b2, sfx2 = bsearch(cg2, 12, KK - above1)
    p24 = lax.shift_left(b1, 12) | b2
    above2 = above1 + sfx2

    compact_hist_pass(NB3, 0, 8, p24, mi)
    pltpu.sync_copy(hist.at[pl.ds(0, NB3)], h3x_hbm.at[pl.ds(wid * NB3, NB3)])
    pltpu.core_barrier(sem, core_axis_name="c")
    plsc.subcore_barrier()
    pltpu.sync_copy(h3x_hbm, h3all.at[pl.ds(0, NW * NB3)])
    for g in range(NB3 // L):
        def ab(w, acc):
            return acc + h3all[pl.ds(w * NB3 + g * L, L)]

        v = lax.fori_loop(0, NW, ab, jnp.zeros((L,), jnp.int32), unroll=4)
        tmp[pl.ds(g * L, L)] = plsc.cumsum(v)
    tot3 = group_prefix(NB3)
    cg3 = make_count_ge(NB3, tot3)
    b3, sfx3 = bsearch(cg3, 8, KK - above2)
    count_greater = above2 + sfx3
    t_bits = lax.shift_left(p24, 8) | b3
    need = KK - count_greater

    def pb(w, acc):
        cnt = h3all[pl.ds(w * NB3 + b3, L)][0]
        return acc + jnp.where(w < wid, cnt, 0)

    base = lax.fori_loop(0, NW, pb, jnp.int32(0))

    tv = jnp.full((L,), t_bits, jnp.int32)
    needv = jnp.full((L,), need, jnp.int32)
    basev = jnp.full((L,), base, jnp.int32)

    def ws(i, carry):
        kv = keys[pl.ds(i * L, L)]
        iv = idxs[pl.ds(i * L, L)]
        gt = kv > tv
        eq = kv == tv
        eqi = jnp.where(eq, 1, 0)
        excl = plsc.cumsum(eqi) - eqi
        rank = basev + carry + excl
        keep = gt | (eq & (rank < needv))
        plsc.store_scatter(xs, [iv], plsc.bitcast(kv, jnp.float32), mask=keep)
        return carry + plsc.all_reduce_population_count(eq)

    lax.fori_loop(0, mi, ws, jnp.zeros((L,), jnp.int32))

    pltpu.sync_copy(xs, out_hbm.at[pl.ds(wid * SHARD, SHARD)])


_fused = pl.kernel(
    _body,
    out_type=(jax.ShapeDtypeStruct((N,), jnp.float32),
              jax.ShapeDtypeStruct((NC * NB12,), jnp.int32),
              jax.ShapeDtypeStruct((NW * NB3,), jnp.int32)),
    mesh=_MESH,
    scratch_types=[pltpu.VMEM((SHARD,), jnp.float32),
                   pltpu.VMEM((NB12,), jnp.int32),
                   pltpu.VMEM((NB12 + L,), jnp.int32),
                   pltpu.VMEM((NW * NB3 + L,), jnp.int32),
                   pltpu.VMEM((SHARD + L,), jnp.int32),
                   pltpu.VMEM((SHARD + L,), jnp.int32),
                   pltpu.VMEM_SHARED((NS * NB12,), jnp.int32),
                   pltpu.VMEM_SHARED((NB12,), jnp.int32),
                   pltpu.SMEM((NB12 // L,), jnp.int32),
                   pltpu.SemaphoreType.REGULAR],
    compiler_params=_SC_PARAMS,
    name="sc_topk_fused",
)


def kernel(x):
    out, _, _ = _fused(x.reshape(-1))
    return out.reshape(x.shape)
